# baseline (speedup 1.0000x reference)
import math
import functools

import jax
import jax.numpy as jnp
from jax import lax
from jax.experimental import pallas as pl
from jax.experimental.pallas import tpu as pltpu
from jax.experimental.pallas import tpu_sc as plsc
from jax._src.pallas import mpmd as _pl_mpmd

CLIP = 1.0
BOUND = 0.1


def _rsqrt16(x):
    xi = plsc.bitcast(x, jnp.int32)
    y = plsc.bitcast(jnp.int32(0x5F3759DF) - (xi >> 1), jnp.float32)
    for _ in range(3):
        y = y * (1.5 - 0.5 * x * y * y)
    return y


def _make_sc_scatter(V, B):
    NW, L = 32, 16
    span = ((V + NW - 1) // NW + L - 1) // L * L
    tail = V - (NW - 1) * span
    assert 0 < tail <= span and tail % L == 0 and span % 8 == 0
    n_grp = B // L

    mesh = plsc.VectorSubcoreMesh(core_axis_name="c", subcore_axis_name="s")

    @functools.partial(
        pl.kernel,
        out_type=jax.ShapeDtypeStruct((V,), jnp.float32),
        mesh=mesh,
        scratch_types=[
            pltpu.VMEM((B,), jnp.int32),
            pltpu.VMEM((B,), jnp.int32),
            pltpu.VMEM((span,), jnp.float32),
        ],
        compiler_params=pltpu.CompilerParams(needs_layout_passes=False),
    )
    def sc_scatter(ids_hbm, cnts_hbm, out_hbm, ids_v, cnts_v, slice_v):
        wid = lax.axis_index("c") * 16 + lax.axis_index("s")
        base = wid * span

        pltpu.sync_copy(ids_hbm, ids_v)
        pltpu.sync_copy(cnts_hbm, cnts_v)

        def scan_body(j, _):
            idv = ids_v[pl.ds(j * L, L)]
            cv = cnts_v[pl.ds(j * L, L)].astype(jnp.float32)
            local = idv - base
            msk = (idv >= base) & (idv < base + span)
            plsc.store_scatter(slice_v, [local], cv, mask=msk)
            return 0

        lax.fori_loop(0, n_grp, scan_body, 0, unroll=4)

        @pl.when(wid < NW - 1)
        def _():
            pltpu.sync_copy(slice_v, out_hbm.at[pl.ds(base, span)])

        @pl.when(wid == NW - 1)
        def _():
            pltpu.sync_copy(
                slice_v.at[pl.ds(0, tail)], out_hbm.at[pl.ds(base, tail)]
            )

    return sc_scatter


def _make_sc_fixup(V, D, B, min_w2):
    NW, L = 32, 16
    CH = B // NW
    assert CH % L == 0 and CH % 8 == 0

    mesh = plsc.VectorSubcoreMesh(core_axis_name="c", subcore_axis_name="s")

    _scratch = [
        pltpu.VMEM((1, CH), jnp.int32),
        pltpu.VMEM((CH,), jnp.float32),
        pltpu.VMEM((CH, D), jnp.float32),
        pltpu.VMEM((CH, D), jnp.float32),
        pltpu.VMEM((CH, D), jnp.float32),
        pltpu.SemaphoreType.DMA,
        pltpu.SemaphoreType.DMA,
        pltpu.SemaphoreType.DMA,
    ]

    def sc_fixup(
        ids_hbm, cf_hbm, w_hbm, g_hbm, out0_hbm, out_hbm,
        ids2d_v, cnt_slab, wbuf, gbuf, obuf, sem_w, sem_g, sem_c,
    ):
        del out0_hbm
        wid = lax.axis_index("c") * 16 + lax.axis_index("s")

        pltpu.sync_copy(ids_hbm.at[pl.ds(wid * CH, CH)], ids2d_v.at[0])
        idx_ref = ids2d_v.at[0]
        ca = pltpu.async_copy(cf_hbm.at[idx_ref], cnt_slab, sem_c)
        cw = pltpu.async_copy(w_hbm.at[idx_ref], wbuf, sem_w)
        cg = pltpu.async_copy(g_hbm.at[idx_ref], gbuf, sem_g)
        ca.wait()
        cw.wait()
        cg.wait()

        lane = lax.iota(jnp.int32, L)

        def sub_body(s, _):
            row = lane + s * L
            cnt16 = cnt_slab[pl.ds(s * L, L)]
            w2 = jnp.zeros((L,), jnp.float32)
            g2 = jnp.zeros((L,), jnp.float32)
            for c in range(D):
                cvec = (jnp.full((L,), c, jnp.int32) + lane) & (D - 1)
                wv = plsc.load_gather(wbuf, [row, cvec])
                gv = plsc.load_gather(gbuf, [row, cvec])
                w2 = w2 + wv * wv
                g2 = g2 + gv * gv
            a2 = jnp.maximum(w2, min_w2)
            ct = cnt16 * (a2 * _rsqrt16(a2))
            ct2 = ct * ct
            mm = jnp.maximum(jnp.maximum(g2, 1e-30), ct2)
            scale = ct * _rsqrt16(mm)
            for c in range(D):
                cvec = (jnp.full((L,), c, jnp.int32) + lane) & (D - 1)
                gv = plsc.load_gather(gbuf, [row, cvec])
                plsc.store_scatter(obuf, [row, cvec], gv * scale)
            return 0

        lax.fori_loop(0, CH // L, sub_body, 0)
        pltpu.async_copy(obuf, out_hbm.at[idx_ref], sem_w).wait()

    return _pl_mpmd._mpmd_map(
        [(mesh, sc_fixup)],
        out_types=jax.ShapeDtypeStruct((V, D), jnp.float32),
        input_output_aliases={4: 0},
        scratch_types=_scratch,
        compiler_params=pltpu.CompilerParams(needs_layout_passes=False),
    )


def _tc_body(min_w2, D, w_ref, g_ref, o_ref):
    w = w_ref[...]
    g = g_ref[...]
    j = jnp.ones((D, D), jnp.float32)
    w2 = jax.lax.dot(w * w, j)
    g2 = jax.lax.dot(g * g, j)
    ct2 = jnp.maximum(w2, min_w2)
    mm = jnp.maximum(jnp.maximum(g2, 1e-30), ct2)
    scale = jnp.sqrt(ct2) * jax.lax.rsqrt(mm)
    o_ref[...] = g * scale


def kernel(w, g, ids, cnts):
    V, D = w.shape
    B = ids.shape[0]
    min_w2 = (CLIP * math.sqrt(D) * BOUND) ** 2

    R = 4000
    nblk = V // R
    assert nblk * R == V and R % 8 == 0

    cnts_full = _make_sc_scatter(V, B)(ids, cnts)

    out0 = pl.pallas_call(
        functools.partial(_tc_body, min_w2, D),
        grid=(nblk,),
        in_specs=[
            pl.BlockSpec((R, D), lambda i: (i, 0)),
            pl.BlockSpec((R, D), lambda i: (i, 0)),
        ],
        out_specs=pl.BlockSpec((R, D), lambda i: (i, 0)),
        out_shape=jax.ShapeDtypeStruct((V, D), jnp.float32),
        compiler_params=pltpu.CompilerParams(
            dimension_semantics=("parallel",)
        ),
    )(w, g)

    return _make_sc_fixup(V, D, B, min_w2)(ids, cnts_full, w, g, out0)


# speedup vs baseline: 1.0678x; 1.0678x over previous
import math
import functools

import jax
import jax.numpy as jnp
from jax import lax
from jax.experimental import pallas as pl
from jax.experimental.pallas import tpu as pltpu
from jax.experimental.pallas import tpu_sc as plsc
from jax._src.pallas import mpmd as _pl_mpmd

CLIP = 1.0
BOUND = 0.1


def _rsqrt16(x):
    xi = plsc.bitcast(x, jnp.int32)
    y = plsc.bitcast(jnp.int32(0x5F3759DF) - (xi >> 1), jnp.float32)
    for _ in range(3):
        y = y * (1.5 - 0.5 * x * y * y)
    return y


def _make_sc_scatter(V, B):
    NW, L = 32, 16
    span = ((V + NW - 1) // NW + L - 1) // L * L
    tail = V - (NW - 1) * span
    assert 0 < tail <= span and tail % L == 0 and span % 8 == 0
    n_grp = B // L

    mesh = plsc.VectorSubcoreMesh(core_axis_name="c", subcore_axis_name="s")

    @functools.partial(
        pl.kernel,
        out_type=jax.ShapeDtypeStruct((V,), jnp.float32),
        mesh=mesh,
        scratch_types=[
            pltpu.VMEM((B,), jnp.int32),
            pltpu.VMEM((B,), jnp.int32),
            pltpu.VMEM((span,), jnp.float32),
        ],
        compiler_params=pltpu.CompilerParams(needs_layout_passes=False),
    )
    def sc_scatter(ids_hbm, cnts_hbm, out_hbm, ids_v, cnts_v, slice_v):
        wid = lax.axis_index("c") * 16 + lax.axis_index("s")
        base = wid * span

        pltpu.sync_copy(ids_hbm, ids_v)
        pltpu.sync_copy(cnts_hbm, cnts_v)

        def scan_body(j, _):
            idv = ids_v[pl.ds(j * L, L)]
            cv = cnts_v[pl.ds(j * L, L)].astype(jnp.float32)
            local = idv - base
            msk = (idv >= base) & (idv < base + span)
            plsc.store_scatter(slice_v, [local], cv, mask=msk)
            return 0

        lax.fori_loop(0, n_grp, scan_body, 0, unroll=4)

        @pl.when(wid < NW - 1)
        def _():
            pltpu.sync_copy(slice_v, out_hbm.at[pl.ds(base, span)])

        @pl.when(wid == NW - 1)
        def _():
            pltpu.sync_copy(
                slice_v.at[pl.ds(0, tail)], out_hbm.at[pl.ds(base, tail)]
            )

    return sc_scatter


def _make_sc_fixup(V, D, B, min_w2):
    NW, L = 32, 16
    CH = B // NW
    assert CH % L == 0 and CH % 8 == 0

    mesh = plsc.VectorSubcoreMesh(core_axis_name="c", subcore_axis_name="s")

    _scratch = [
        pltpu.VMEM((1, CH), jnp.int32),
        pltpu.VMEM((CH,), jnp.float32),
        pltpu.VMEM((CH, D), jnp.float32),
        pltpu.VMEM((CH, D), jnp.float32),
        pltpu.VMEM((CH, D), jnp.float32),
        pltpu.SemaphoreType.DMA,
        pltpu.SemaphoreType.DMA,
        pltpu.SemaphoreType.DMA,
    ]

    def sc_fixup(
        ids_hbm, cf_hbm, w_hbm, g_hbm, out0_hbm, out_hbm,
        ids2d_v, cnt_slab, wbuf, gbuf, obuf, sem_w, sem_g, sem_c,
    ):
        del out0_hbm
        wid = lax.axis_index("c") * 16 + lax.axis_index("s")

        pltpu.sync_copy(ids_hbm.at[pl.ds(wid * CH, CH)], ids2d_v.at[0])
        idx_ref = ids2d_v.at[0]
        ca = pltpu.async_copy(cf_hbm.at[idx_ref], cnt_slab, sem_c)
        cw = pltpu.async_copy(w_hbm.at[idx_ref], wbuf, sem_w)
        cg = pltpu.async_copy(g_hbm.at[idx_ref], gbuf, sem_g)
        ca.wait()
        cw.wait()
        cg.wait()

        lane = lax.iota(jnp.int32, L)

        def sub_body(s, _):
            row = lane + s * L
            cnt16 = cnt_slab[pl.ds(s * L, L)]
            w2 = jnp.zeros((L,), jnp.float32)
            g2 = jnp.zeros((L,), jnp.float32)
            for c in range(D):
                cvec = (jnp.full((L,), c, jnp.int32) + lane) & (D - 1)
                wv = plsc.load_gather(wbuf, [row, cvec])
                gv = plsc.load_gather(gbuf, [row, cvec])
                w2 = w2 + wv * wv
                g2 = g2 + gv * gv
            a2 = jnp.maximum(w2, min_w2)
            ct = cnt16 * (a2 * _rsqrt16(a2))
            ct2 = ct * ct
            mm = jnp.maximum(jnp.maximum(g2, 1e-30), ct2)
            scale = ct * _rsqrt16(mm)
            for c in range(D):
                cvec = (jnp.full((L,), c, jnp.int32) + lane) & (D - 1)
                gv = plsc.load_gather(gbuf, [row, cvec])
                plsc.store_scatter(obuf, [row, cvec], gv * scale)
            return 0

        lax.fori_loop(0, CH // L, sub_body, 0)
        pltpu.async_copy(obuf, out_hbm.at[idx_ref], sem_w).wait()

    return _pl_mpmd._mpmd_map(
        [(mesh, sc_fixup)],
        out_types=jax.ShapeDtypeStruct((V, D), jnp.float32),
        input_output_aliases={4: 0},
        scratch_types=_scratch,
        compiler_params=pltpu.CompilerParams(needs_layout_passes=False),
    )


def _tc_body(min_w2, D, w_ref, g_ref, o_ref):
    w = w_ref[...]
    g = g_ref[...]
    j = jnp.ones((D, D), jnp.float32)
    w2 = jax.lax.dot(w * w, j)
    g2 = jax.lax.dot(g * g, j)
    ct2 = jnp.maximum(w2, min_w2)
    mm = jnp.maximum(jnp.maximum(g2, 1e-30), ct2)
    scale = jnp.sqrt(ct2) * jax.lax.rsqrt(mm)
    o_ref[...] = g * scale


def kernel(w, g, ids, cnts):
    V, D = w.shape
    B = ids.shape[0]
    min_w2 = (CLIP * math.sqrt(D) * BOUND) ** 2

    R = 10000
    nblk = V // R
    assert nblk * R == V and R % 8 == 0

    cnts_full = _make_sc_scatter(V, B)(ids, cnts)

    out0 = pl.pallas_call(
        functools.partial(_tc_body, min_w2, D),
        grid=(nblk,),
        in_specs=[
            pl.BlockSpec((R, D), lambda i: (i, 0)),
            pl.BlockSpec((R, D), lambda i: (i, 0)),
        ],
        out_specs=pl.BlockSpec((R, D), lambda i: (i, 0)),
        out_shape=jax.ShapeDtypeStruct((V, D), jnp.float32),
        compiler_params=pltpu.CompilerParams(
            dimension_semantics=("parallel",)
        ),
    )(w, g)

    return _make_sc_fixup(V, D, B, min_w2)(ids, cnts_full, w, g, out0)
